# baseline (speedup 1.0000x reference)
import jax
import jax.numpy as jnp
from jax import lax
from jax.experimental import pallas as pl
from jax.experimental.pallas import tpu as pltpu
from jax.experimental.pallas import tpu_sc as plsc

B = 16384
NF = 128
HALF = 384
TFLAT = 2 * HALF
NC = 2
NS = 16
L = 16
NW = NC * NS
ROWS_PER_W = B // NW
NCH = NF // L


def _sc_body(x_hbm, e_hbm, v_hbm, t_hbm, w_hbm, b_hbm, o_hbm,
             x_v, e_v, v_v, o_v, t_v, c_v, w_v, b_v,
             sem_x, sem_x2, sem_x3, sem_x4, sem_s):
  wid = lax.axis_index("s") * NC + lax.axis_index("c")
  base = wid * ROWS_PER_W

  quart = ROWS_PER_W // 4
  cp_x = [pltpu.async_copy(x_hbm.at[pl.ds(base + q * quart, quart), :],
                           x_v.at[pl.ds(q * quart, quart), :], sx)
          for q, sx in enumerate((sem_x, sem_x2, sem_x3, sem_x4))]
  cp = [pltpu.async_copy(e_hbm.at[pl.ds(base, ROWS_PER_W)], e_v, sem_s),
        pltpu.async_copy(v_hbm.at[pl.ds(base, ROWS_PER_W)], v_v, sem_s),
        pltpu.async_copy(t_hbm, t_v, sem_s),
        pltpu.async_copy(w_hbm, w_v, sem_s),
        pltpu.async_copy(b_hbm, b_v.at[pl.ds(0, 2)], sem_s)]
  for c in cp:
    c.wait()

  lanes = lax.iota(jnp.int32, L)

  dnums = lax.GatherDimensionNumbers(
      offset_dims=(), collapsed_slice_dims=(0,), start_index_map=(0,))

  def permute(vec, idx):
    return lax.gather(vec, idx[:, None], dimension_numbers=dnums,
                      slice_sizes=(1,),
                      mode=lax.GatherScatterMode.PROMISE_IN_BOUNDS)

  def lane_sum(vec):
    for sh in (1, 2, 4, 8):
      vec = vec + permute(vec, lanes ^ sh)
    return vec

  for g in range(6):
    parts = [t_v[pl.ds(g * 128 + L * j, L)] for j in range(128 // L)]
    tot = ((parts[0] + parts[1]) + (parts[2] + parts[3])) + \
          ((parts[4] + parts[5]) + (parts[6] + parts[7]))
    for sh in (2, 4, 8):
      tot = tot + permute(tot, lanes ^ sh)
    mean = tot * (1.0 / 64.0)
    for j in range(128 // L):
      c_v[pl.ds(g * 128 + L * j, L)] = parts[j] - mean

  bvec = b_v[pl.ds(0, L)]
  b0 = bvec[0]
  db = bvec[1] - b0
  w0s = [w_v[pl.ds(c * L, L)] for c in range(NCH)]
  dws = [w_v[pl.ds(NF + c * L, L)] - w0s[c] for c in range(NCH)]

  def group_body(g, carry):
    row0 = g * L
    sl = pl.ds(row0, L)
    e = e_v[sl]
    dense0 = jnp.zeros((L,), jnp.float32)
    dense1 = jnp.zeros((L,), jnp.float32)
    for r in range(L):
      row = row0 + r
      xs = [x_v[row, pl.ds(c * L, L)] for c in range(NCH)]
      acc0a = xs[0] * w0s[0]
      acc0b = xs[4] * w0s[4]
      acc1a = xs[0] * dws[0]
      acc1b = xs[4] * dws[4]
      for c in range(1, NCH // 2):
        acc0a += xs[c] * w0s[c]
        acc0b += xs[c + 4] * w0s[c + 4]
        acc1a += xs[c] * dws[c]
        acc1b += xs[c + 4] * dws[c + 4]
      acc0 = lane_sum(acc0a + acc0b)
      acc1 = lane_sum(acc1a + acc1b)
      dense0 = jnp.where(lanes == r, acc0, dense0)
      dense1 = jnp.where(lanes == r, acc1, dense1)

    v = v_v[sl]
    neg = v >= HALF
    jj = jnp.where(neg, v - HALF, v)
    sgn = jnp.where(neg, -1.0, 1.0)
    jj2 = jj + jj
    g0 = plsc.load_gather(c_v, [jj2])
    g1 = plsc.load_gather(c_v, [jj2 + 1])
    o_v[sl] = dense0 + b0 + e * (dense1 + db) + sgn * (g0 + e * (g1 - g0))
    return carry

  ngroups = ROWS_PER_W // L
  gq = ngroups // 4
  for q in range(4):
    cp_x[q].wait()
    lax.fori_loop(q * gq, (q + 1) * gq, group_body, 0)

  pltpu.sync_copy(o_v, o_hbm.at[pl.ds(base, ROWS_PER_W)])


_sc_kernel = pl.kernel(
    _sc_body,
    out_type=jax.ShapeDtypeStruct((B,), jnp.float32),
    mesh=plsc.VectorSubcoreMesh(core_axis_name="c", subcore_axis_name="s"),
    compiler_params=pltpu.CompilerParams(needs_layout_passes=False),
    scratch_types=[
        pltpu.VMEM((ROWS_PER_W, NF), jnp.float32),
        pltpu.VMEM((ROWS_PER_W,), jnp.float32),
        pltpu.VMEM((ROWS_PER_W,), jnp.int32),
        pltpu.VMEM((ROWS_PER_W,), jnp.float32),
        pltpu.VMEM((TFLAT,), jnp.float32),
        pltpu.VMEM((TFLAT,), jnp.float32),
        pltpu.VMEM((2 * NF,), jnp.float32),
        pltpu.VMEM((L,), jnp.float32),
        pltpu.SemaphoreType.DMA,
        pltpu.SemaphoreType.DMA,
        pltpu.SemaphoreType.DMA,
        pltpu.SemaphoreType.DMA,
        pltpu.SemaphoreType.DMA,
    ],
)


@jax.jit
def kernel(x, earliness, pst_values, pst_lengths, W, b, pst_param):
  del pst_lengths
  return _sc_kernel(x, earliness, pst_values.astype(jnp.int32),
                    pst_param.reshape(TFLAT), W.reshape(2 * NF), b)


# speedup vs baseline: 1.2078x; 1.2078x over previous
import jax
import jax.numpy as jnp
from jax import lax
from jax.experimental import pallas as pl
from jax.experimental.pallas import tpu as pltpu
from jax.experimental.pallas import tpu_sc as plsc

B = 16384
NF = 128
HALF = 384
TFLAT = 2 * HALF
NC = 2
NS = 16
L = 16
NW = NC * NS
ROWS_PER_W = B // NW
NCH = NF // L


def _sc_body(x_hbm, e_hbm, v_hbm, t_hbm, w_hbm, b_hbm, o_hbm,
             x_v, e_v, v_v, o_v, t_v, c_v, w_v, b_v,
             sem_x, sem_x2, sem_s):
  wid = lax.axis_index("s") * NC + lax.axis_index("c")
  base = wid * ROWS_PER_W

  half = ROWS_PER_W // 2
  cp_x0 = pltpu.async_copy(x_hbm.at[pl.ds(base, half), :],
                           x_v.at[pl.ds(0, half), :], sem_x)
  cp_x1 = pltpu.async_copy(x_hbm.at[pl.ds(base + half, half), :],
                           x_v.at[pl.ds(half, half), :], sem_x2)
  cp = [pltpu.async_copy(e_hbm.at[pl.ds(base, ROWS_PER_W)], e_v, sem_s),
        pltpu.async_copy(v_hbm.at[pl.ds(base, ROWS_PER_W)], v_v, sem_s),
        pltpu.async_copy(t_hbm, t_v, sem_s),
        pltpu.async_copy(w_hbm, w_v, sem_s),
        pltpu.async_copy(b_hbm, b_v.at[pl.ds(0, 2)], sem_s)]
  for c in cp:
    c.wait()

  lanes = lax.iota(jnp.int32, L)

  dnums = lax.GatherDimensionNumbers(
      offset_dims=(), collapsed_slice_dims=(0,), start_index_map=(0,))

  def permute(vec, idx):
    return lax.gather(vec, idx[:, None], dimension_numbers=dnums,
                      slice_sizes=(1,),
                      mode=lax.GatherScatterMode.PROMISE_IN_BOUNDS)

  def lane_sum(vec):
    for sh in (1, 2, 4, 8):
      vec = vec + permute(vec, lanes ^ sh)
    return vec

  for g in range(6):
    parts = [t_v[pl.ds(g * 128 + L * j, L)] for j in range(128 // L)]
    tot = ((parts[0] + parts[1]) + (parts[2] + parts[3])) + \
          ((parts[4] + parts[5]) + (parts[6] + parts[7]))
    for sh in (2, 4, 8):
      tot = tot + permute(tot, lanes ^ sh)
    mean = tot * (1.0 / 64.0)
    for j in range(128 // L):
      c_v[pl.ds(g * 128 + L * j, L)] = parts[j] - mean

  bvec = b_v[pl.ds(0, L)]
  b0 = bvec[0]
  db = bvec[1] - b0
  w0s = [w_v[pl.ds(c * L, L)] for c in range(NCH)]
  dws = [w_v[pl.ds(NF + c * L, L)] - w0s[c] for c in range(NCH)]

  def group_body(g, carry):
    row0 = g * L
    sl = pl.ds(row0, L)
    e = e_v[sl]
    dense0 = jnp.zeros((L,), jnp.float32)
    dense1 = jnp.zeros((L,), jnp.float32)

    def row_partials(r):
      row = row0 + r
      xs = [x_v[row, pl.ds(c * L, L)] for c in range(NCH)]
      acc0a = xs[0] * w0s[0]
      acc0b = xs[4] * w0s[4]
      acc1a = xs[0] * dws[0]
      acc1b = xs[4] * dws[4]
      for c in range(1, NCH // 2):
        acc0a += xs[c] * w0s[c]
        acc0b += xs[c + 4] * w0s[c + 4]
        acc1a += xs[c] * dws[c]
        acc1b += xs[c + 4] * dws[c + 4]
      return acc0a + acc0b, acc1a + acc1b

    for i in range(L // 2):
      a0, a1 = row_partials(2 * i)
      b0, b1 = row_partials(2 * i + 1)
      odd = (lanes & 1) != 0
      m0 = jnp.where(odd, b0 + permute(b0, lanes ^ 1), a0 + permute(a0, lanes ^ 1))
      m1 = jnp.where(odd, b1 + permute(b1, lanes ^ 1), a1 + permute(a1, lanes ^ 1))
      for sh in (2, 4, 8):
        m0 = m0 + permute(m0, lanes ^ sh)
        m1 = m1 + permute(m1, lanes ^ sh)
      sel = (lanes | 1) == (2 * i + 1)
      dense0 = jnp.where(sel, m0, dense0)
      dense1 = jnp.where(sel, m1, dense1)

    v = v_v[sl]
    neg = v >= HALF
    jj = jnp.where(neg, v - HALF, v)
    sgn = jnp.where(neg, -1.0, 1.0)
    jj2 = jj + jj
    g0 = plsc.load_gather(c_v, [jj2])
    g1 = plsc.load_gather(c_v, [jj2 + 1])
    o_v[sl] = dense0 + b0 + e * (dense1 + db) + sgn * (g0 + e * (g1 - g0))
    return carry

  ngroups = ROWS_PER_W // L
  cp_x0.wait()
  lax.fori_loop(0, ngroups // 2, group_body, 0)
  cp_x1.wait()
  lax.fori_loop(ngroups // 2, ngroups, group_body, 0)

  pltpu.sync_copy(o_v, o_hbm.at[pl.ds(base, ROWS_PER_W)])


_sc_kernel = pl.kernel(
    _sc_body,
    out_type=jax.ShapeDtypeStruct((B,), jnp.float32),
    mesh=plsc.VectorSubcoreMesh(core_axis_name="c", subcore_axis_name="s"),
    compiler_params=pltpu.CompilerParams(needs_layout_passes=False),
    scratch_types=[
        pltpu.VMEM((ROWS_PER_W, NF), jnp.float32),
        pltpu.VMEM((ROWS_PER_W,), jnp.float32),
        pltpu.VMEM((ROWS_PER_W,), jnp.int32),
        pltpu.VMEM((ROWS_PER_W,), jnp.float32),
        pltpu.VMEM((TFLAT,), jnp.float32),
        pltpu.VMEM((TFLAT,), jnp.float32),
        pltpu.VMEM((2 * NF,), jnp.float32),
        pltpu.VMEM((L,), jnp.float32),
        pltpu.SemaphoreType.DMA,
        pltpu.SemaphoreType.DMA,
        pltpu.SemaphoreType.DMA,
    ],
)


@jax.jit
def kernel(x, earliness, pst_values, pst_lengths, W, b, pst_param):
  del pst_lengths
  return _sc_kernel(x, earliness, pst_values.astype(jnp.int32),
                    pst_param.reshape(TFLAT), W.reshape(2 * NF), b)
